# baseline (speedup 1.0000x reference)
import functools

import jax
import jax.numpy as jnp
from jax import lax
from jax.experimental import pallas as pl
from jax.experimental.pallas import tpu as pltpu
from jax.experimental.pallas import tpu_sc as plsc

B = 1024
L = 200
EMB = 64
N = B * L
NC, NS = 2, 16
NW = NC * NS
PER_W = N // NW
CHUNK = 128
NCHUNK = PER_W // CHUNK
POSF = L * EMB


@functools.partial(
    pl.kernel,
    out_type=jax.ShapeDtypeStruct((N, EMB), jnp.float32),
    mesh=plsc.VectorSubcoreMesh(core_axis_name="c", subcore_axis_name="s"),
    compiler_params=pltpu.CompilerParams(use_tc_tiling_on_sc=False),
    scratch_types=[
        pltpu.VMEM((NCHUNK, CHUNK), jnp.int32),
        pltpu.VMEM((2 * POSF,), jnp.float32),
        pltpu.VMEM((CHUNK, EMB), jnp.float32),
        pltpu.SemaphoreType.DMA,
    ],
)
def _emb_lookup(idx_hbm, pos_hbm, table_hbm, out_hbm, idx_v, pos_v, rows_v, sem):
    wid = lax.axis_index("s") * NC + lax.axis_index("c")
    base = wid * PER_W
    pltpu.sync_copy(idx_hbm.at[wid], idx_v)
    pltpu.sync_copy(pos_hbm, pos_v)

    def chunk_body(c, carry):
        pltpu.async_copy(table_hbm.at[idx_v.at[c]], rows_v, sem).wait()
        phase = ((c * CHUNK) % L) * EMB

        def row_body(i, carry2):
            off = phase + i * EMB
            for j in range(EMB // 16):
                s = j * 16
                rows_v[i, pl.ds(s, 16)] = (
                    rows_v[i, pl.ds(s, 16)] + pos_v[pl.ds(off + s, 16)]
                )
            return carry2

        lax.fori_loop(0, CHUNK, row_body, 0)
        pltpu.sync_copy(rows_v, out_hbm.at[pl.ds(base + c * CHUNK, CHUNK)])
        return carry

    lax.fori_loop(0, NCHUNK, chunk_body, 0)


def kernel(x, table, pos_emb):
    idx = x.astype(jnp.int32).reshape(NW, NCHUNK, CHUNK)
    pos_flat = pos_emb.astype(jnp.float32).reshape(POSF)
    pos_d = jnp.concatenate([pos_flat, pos_flat])
    out = _emb_lookup(idx, pos_d, table)
    return out.reshape(B, L, EMB)


# speedup vs baseline: 1.2061x; 1.2061x over previous
import functools

import jax
import jax.numpy as jnp
from jax import lax
from jax.experimental import pallas as pl
from jax.experimental.pallas import tpu as pltpu
from jax.experimental.pallas import tpu_sc as plsc

B = 1024
L = 200
EMB = 64
N = B * L
NC, NS = 2, 16
NW = NC * NS
CHUNK = 128
CPL = B // CHUNK
NCHUNK = L * CPL // NW
POSF = L * EMB
NBUF = 5
AHEAD = 2
RU = 8


@functools.partial(
    pl.kernel,
    out_type=jax.ShapeDtypeStruct((N, EMB), jnp.float32),
    mesh=plsc.VectorSubcoreMesh(core_axis_name="c", subcore_axis_name="s"),
    compiler_params=pltpu.CompilerParams(use_tc_tiling_on_sc=False),
    scratch_types=(
        [
            pltpu.VMEM((NCHUNK, CHUNK), jnp.int32),
            pltpu.VMEM((NCHUNK, CHUNK), jnp.int32),
            pltpu.VMEM((POSF,), jnp.float32),
        ]
        + [pltpu.VMEM((CHUNK, EMB), jnp.float32) for _ in range(NBUF)]
        + [pltpu.SemaphoreType.DMA for _ in range(2 * NBUF)]
    ),
)
def _emb_lookup(idx_hbm, oidx_hbm, pos_hbm, table_hbm, out_hbm, *refs):
    idx_v, oidx_v, pos_v = refs[0], refs[1], refs[2]
    rows = refs[3:3 + NBUF]
    sem_g = refs[3 + NBUF:3 + 2 * NBUF]
    sem_w = refs[3 + 2 * NBUF:3 + 3 * NBUF]

    wid = lax.axis_index("s") * NC + lax.axis_index("c")
    pltpu.sync_copy(idx_hbm.at[wid], idx_v)
    pltpu.sync_copy(oidx_hbm.at[wid], oidx_v)
    pltpu.sync_copy(pos_hbm, pos_v)
    for b in range(AHEAD):
        pltpu.async_copy(table_hbm.at[idx_v.at[b]], rows[b], sem_g[b])
    gbase = wid * NCHUNK

    def outer(t, carry):
        for b in range(NBUF):
            c = t * NBUF + b
            pltpu.make_async_copy(
                table_hbm.at[idx_v.at[c]], rows[b], sem_g[b]
            ).wait()
            l = (gbase + c) // CPL
            pvec = [pos_v[pl.ds(l * EMB + 16 * j, 16)] for j in range(4)]

            def row_body(i, carry2, b=b, pvec=pvec):
                for r in range(RU):
                    ii = i * RU + r
                    for j in range(4):
                        s = 16 * j
                        rows[b][ii, pl.ds(s, 16)] = (
                            rows[b][ii, pl.ds(s, 16)] + pvec[j]
                        )
                return carry2

            lax.fori_loop(0, CHUNK // RU, row_body, 0)
            pltpu.async_copy(rows[b], out_hbm.at[oidx_v.at[c]], sem_w[b])

            bn = (b + AHEAD) % NBUF
            cn = c + AHEAD

            @pl.when(cn < NCHUNK)
            def _issue(bn=bn, cn=cn, c=c):
                @pl.when(c >= NBUF - AHEAD)
                def _drain():
                    pltpu.make_async_copy(
                        rows[bn], out_hbm.at[oidx_v.at[c]], sem_w[bn]
                    ).wait()

                pltpu.async_copy(table_hbm.at[idx_v.at[cn]], rows[bn], sem_g[bn])

        return carry

    lax.fori_loop(0, NCHUNK // NBUF, outer, 0)
    for b in range(NBUF):
        pltpu.make_async_copy(
            rows[b], out_hbm.at[oidx_v.at[b]], sem_w[b]
        ).wait()


def kernel(x, table, pos_emb):
    idx = x.astype(jnp.int32).T.reshape(NW, NCHUNK, CHUNK)
    oidx = (
        (jnp.arange(B, dtype=jnp.int32) * L)[None, :]
        + jnp.arange(L, dtype=jnp.int32)[:, None]
    ).reshape(NW, NCHUNK, CHUNK)
    pos_flat = pos_emb.astype(jnp.float32).reshape(POSF)
    out = _emb_lookup(idx, oidx, pos_flat, table)
    return out.reshape(B, L, EMB)
